# Initial kernel scaffold; baseline (speedup 1.0000x reference)
import jax
import jax.numpy as jnp
from jax.experimental import pallas as pl


def kernel(x, router_w, w1, w2, w3):
    raise NotImplementedError("write your pallas kernel here")


# speedup vs baseline: 5.9961x; 5.9961x over previous
import functools

import jax
import jax.numpy as jnp
from jax import lax
from jax.experimental import pallas as pl
from jax.experimental.pallas import tpu as pltpu

NUM_EXPERTS = 64
MODEL_DIM = 768
EXPERT_DIM = 768
TOKENS = 4096

PAD8 = 8
TOKENS_PAD = TOKENS + NUM_EXPERTS * PAD8
BLK = 128
MAX_TILES = NUM_EXPERTS - 1 + TOKENS // BLK


def _router_body(x_ref, rw_ref, out_ref):
    logits = lax.dot_general(
        x_ref[...], rw_ref[...],
        dimension_numbers=(((1,), (1,)), ((), ())),
        preferred_element_type=jnp.float32,
    )
    m = jnp.max(logits, axis=1, keepdims=True)
    p = jnp.exp(logits - m)
    p = p / jnp.sum(p, axis=1, keepdims=True)
    pm = jnp.max(p, axis=1, keepdims=True)
    ids = lax.broadcasted_iota(jnp.int32, p.shape, 1)
    sel = jnp.min(jnp.where(p >= pm, ids, NUM_EXPERTS), axis=1, keepdims=True)
    out_ref[...] = sel


def _router(x, router_w):
    bt = 512
    return pl.pallas_call(
        _router_body,
        grid=(TOKENS // bt,),
        in_specs=[
            pl.BlockSpec((bt, MODEL_DIM), lambda t: (t, 0)),
            pl.BlockSpec((NUM_EXPERTS, MODEL_DIM), lambda t: (0, 0)),
        ],
        out_specs=pl.BlockSpec((bt, 1), lambda t: (t, 0)),
        out_shape=jax.ShapeDtypeStruct((TOKENS, 1), jnp.int32),
    )(x, router_w).reshape(TOKENS)


def _gemm_body(e_ref, c_ref, lo_ref, hi_ref,
               xs_ref, w1_ref, w3_ref, w2_ref, out_ref):
    t = pl.program_id(0)
    c = pl.multiple_of(c_ref[t], PAD8)
    lo = lo_ref[t]
    hi = hi_ref[t]

    @pl.when(hi > lo)
    def _():
        xb = xs_ref[pl.ds(c, BLK), :]
        w1e = w1_ref[0]
        w3e = w3_ref[0]
        w2e = w2_ref[0]
        nt = (((1,), (1,)), ((), ()))
        a = lax.dot_general(xb, w1e, nt, preferred_element_type=jnp.float32)
        b = lax.dot_general(xb, w3e, nt, preferred_element_type=jnp.float32)
        h = (a * jax.nn.sigmoid(a)) * b
        ob = lax.dot_general(h, w2e, nt, preferred_element_type=jnp.float32)
        rows = lax.broadcasted_iota(jnp.int32, (BLK, 1), 0)
        keep = (rows >= lo) & (rows < hi)
        cur = out_ref[pl.ds(c, BLK), :]
        out_ref[pl.ds(c, BLK), :] = jnp.where(keep, ob, cur)


def _grouped_gemm(e_arr, c_arr, lo_arr, hi_arr, xs, w1, w3, w2):
    spec = pltpu.PrefetchScalarGridSpec(
        num_scalar_prefetch=4,
        grid=(MAX_TILES,),
        in_specs=[
            pl.BlockSpec((TOKENS_PAD, MODEL_DIM), lambda t, e, c, l, h: (0, 0)),
            pl.BlockSpec((1, EXPERT_DIM, MODEL_DIM),
                         lambda t, e, c, l, h: (e[t], 0, 0)),
            pl.BlockSpec((1, EXPERT_DIM, MODEL_DIM),
                         lambda t, e, c, l, h: (e[t], 0, 0)),
            pl.BlockSpec((1, MODEL_DIM, EXPERT_DIM),
                         lambda t, e, c, l, h: (e[t], 0, 0)),
        ],
        out_specs=pl.BlockSpec((TOKENS_PAD, MODEL_DIM),
                               lambda t, e, c, l, h: (0, 0)),
    )
    return pl.pallas_call(
        _gemm_body,
        grid_spec=spec,
        out_shape=jax.ShapeDtypeStruct((TOKENS_PAD, MODEL_DIM), jnp.float32),
        compiler_params=pltpu.CompilerParams(
            dimension_semantics=("arbitrary",)),
    )(e_arr, c_arr, lo_arr, hi_arr, xs, w1, w3, w2)


def kernel(x, router_w, w1, w2, w3):
    expert = _router(x, router_w)

    order = jnp.argsort(expert)
    sorted_e = jnp.take(expert, order)
    counts = jnp.zeros((NUM_EXPERTS,), jnp.int32).at[expert].add(1)
    pcounts = (counts + (PAD8 - 1)) // PAD8 * PAD8
    poff = jnp.cumsum(pcounts) - pcounts
    coff = jnp.cumsum(counts) - counts
    i_range = jnp.arange(TOKENS, dtype=jnp.int32)
    rank = i_range - jnp.take(coff, sorted_e)
    slot = jnp.take(poff, sorted_e) + rank
    gather_idx = jnp.zeros((TOKENS_PAD,), jnp.int32).at[slot].set(order)
    token_slot = jnp.zeros((TOKENS,), jnp.int32).at[order].set(slot)

    tiles_pe = (counts + (BLK - 1)) // BLK
    tcum = jnp.cumsum(tiles_pe)
    total = tcum[NUM_EXPERTS - 1]
    t_range = jnp.arange(MAX_TILES, dtype=jnp.int32)
    e_raw = jnp.searchsorted(tcum, t_range, side="right").astype(jnp.int32)
    e_cl = jnp.minimum(e_raw, NUM_EXPERTS - 1)
    i_t = t_range - (jnp.take(tcum, e_cl) - jnp.take(tiles_pe, e_cl))
    s_t = jnp.take(poff, e_cl) + i_t * BLK
    valid = jnp.clip(jnp.take(counts, e_cl) - i_t * BLK, 0, BLK)
    c_t = jnp.minimum(s_t, TOKENS_PAD - BLK)
    lo = s_t - c_t
    hi = lo + valid
    is_pad = t_range >= total
    e_last = sorted_e[TOKENS - 1]
    e_arr = jnp.where(is_pad, e_last, e_cl).astype(jnp.int32)
    c_arr = jnp.where(is_pad, 0, c_t).astype(jnp.int32)
    lo_arr = jnp.where(is_pad, 0, lo).astype(jnp.int32)
    hi_arr = jnp.where(is_pad, 0, hi).astype(jnp.int32)

    xs = jnp.take(x, gather_idx, axis=0)
    os_ = _grouped_gemm(e_arr, c_arr, lo_arr, hi_arr, xs, w1, w3, w2)
    return jnp.take(os_, token_slot, axis=0)


# speedup vs baseline: 6.4848x; 1.0815x over previous
import functools

import jax
import jax.numpy as jnp
from jax import lax
from jax.experimental import pallas as pl
from jax.experimental.pallas import tpu as pltpu
from jax.experimental.pallas import tpu_sc as plsc

NUM_EXPERTS = 64
MODEL_DIM = 768
EXPERT_DIM = 768
TOKENS = 4096

PAD8 = 8
TOKENS_PAD = TOKENS + NUM_EXPERTS * PAD8
BLK = 128
MAX_TILES = NUM_EXPERTS - 1 + TOKENS // BLK


def _router_body(x_ref, rw_ref, out_ref):
    logits = lax.dot_general(
        x_ref[...], rw_ref[...],
        dimension_numbers=(((1,), (1,)), ((), ())),
        preferred_element_type=jnp.float32,
    )
    m = jnp.max(logits, axis=1, keepdims=True)
    p = jnp.exp(logits - m)
    p = p / jnp.sum(p, axis=1, keepdims=True)
    pm = jnp.max(p, axis=1, keepdims=True)
    ids = lax.broadcasted_iota(jnp.int32, p.shape, 1)
    sel = jnp.min(jnp.where(p >= pm, ids, NUM_EXPERTS), axis=1, keepdims=True)
    out_ref[...] = sel


def _router(x, router_w):
    bt = 512
    return pl.pallas_call(
        _router_body,
        grid=(TOKENS // bt,),
        in_specs=[
            pl.BlockSpec((bt, MODEL_DIM), lambda t: (t, 0)),
            pl.BlockSpec((NUM_EXPERTS, MODEL_DIM), lambda t: (0, 0)),
        ],
        out_specs=pl.BlockSpec((bt, 1), lambda t: (t, 0)),
        out_shape=jax.ShapeDtypeStruct((TOKENS, 1), jnp.int32),
    )(x, router_w).reshape(TOKENS)


_SC_CORES = 2
_SC_SUBCORES = 16
_SC_WORKERS = _SC_CORES * _SC_SUBCORES
_SC_CHUNK = 128


def _sc_gather_rows(table, idx):
    n_rows = idx.shape[0]
    assert n_rows % _SC_CHUNK == 0
    n_chunks = n_rows // _SC_CHUNK
    rounds = (n_chunks + _SC_WORKERS - 1) // _SC_WORKERS
    mesh = plsc.VectorSubcoreMesh(core_axis_name="c", subcore_axis_name="s")

    @functools.partial(
        pl.kernel,
        mesh=mesh,
        out_type=jax.ShapeDtypeStruct((n_rows, MODEL_DIM), jnp.float32),
        scratch_types=[
            pltpu.VMEM((_SC_CHUNK,), jnp.int32),
            pltpu.VMEM((_SC_CHUNK, MODEL_DIM), jnp.float32),
            pltpu.SemaphoreType.DMA,
        ],
    )
    def k(table_hbm, idx_hbm, out_hbm, idx_v, rows_v, sem):
        wid = lax.axis_index("s") * _SC_CORES + lax.axis_index("c")
        for j in range(rounds):
            chunk = wid + j * _SC_WORKERS

            @pl.when(chunk < n_chunks)
            def _():
                base = chunk * _SC_CHUNK
                pltpu.sync_copy(idx_hbm.at[pl.ds(base, _SC_CHUNK)], idx_v)
                pltpu.async_copy(table_hbm.at[idx_v], rows_v, sem).wait()
                pltpu.sync_copy(rows_v, out_hbm.at[pl.ds(base, _SC_CHUNK)])

    return k(table, idx)


def _gemm_body(e_ref, c_ref, lo_ref, hi_ref,
               xs_ref, w1_ref, w3_ref, w2_ref, out_ref):
    t = pl.program_id(0)
    c = pl.multiple_of(c_ref[t], PAD8)
    lo = lo_ref[t]
    hi = hi_ref[t]

    @pl.when(hi > lo)
    def _():
        xb = xs_ref[pl.ds(c, BLK), :]
        w1e = w1_ref[0]
        w3e = w3_ref[0]
        w2e = w2_ref[0]
        nt = (((1,), (1,)), ((), ()))
        a = lax.dot_general(xb, w1e, nt, preferred_element_type=jnp.float32)
        b = lax.dot_general(xb, w3e, nt, preferred_element_type=jnp.float32)
        h = (a * jax.nn.sigmoid(a)) * b
        ob = lax.dot_general(h, w2e, nt, preferred_element_type=jnp.float32)
        rows = lax.broadcasted_iota(jnp.int32, (BLK, 1), 0)
        keep = (rows >= lo) & (rows < hi)
        cur = out_ref[pl.ds(c, BLK), :]
        out_ref[pl.ds(c, BLK), :] = jnp.where(keep, ob, cur)


def _grouped_gemm(e_arr, c_arr, lo_arr, hi_arr, xs, w1, w3, w2):
    spec = pltpu.PrefetchScalarGridSpec(
        num_scalar_prefetch=4,
        grid=(MAX_TILES,),
        in_specs=[
            pl.BlockSpec((TOKENS_PAD, MODEL_DIM), lambda t, e, c, l, h: (0, 0)),
            pl.BlockSpec((1, EXPERT_DIM, MODEL_DIM),
                         lambda t, e, c, l, h: (e[t], 0, 0)),
            pl.BlockSpec((1, EXPERT_DIM, MODEL_DIM),
                         lambda t, e, c, l, h: (e[t], 0, 0)),
            pl.BlockSpec((1, MODEL_DIM, EXPERT_DIM),
                         lambda t, e, c, l, h: (e[t], 0, 0)),
        ],
        out_specs=pl.BlockSpec((TOKENS_PAD, MODEL_DIM),
                               lambda t, e, c, l, h: (0, 0)),
    )
    return pl.pallas_call(
        _gemm_body,
        grid_spec=spec,
        out_shape=jax.ShapeDtypeStruct((TOKENS_PAD, MODEL_DIM), jnp.float32),
        compiler_params=pltpu.CompilerParams(
            dimension_semantics=("arbitrary",)),
    )(e_arr, c_arr, lo_arr, hi_arr, xs, w1, w3, w2)


def kernel(x, router_w, w1, w2, w3):
    expert = _router(x, router_w)

    order = jnp.argsort(expert)
    sorted_e = jnp.take(expert, order)
    counts = jnp.zeros((NUM_EXPERTS,), jnp.int32).at[expert].add(1)
    pcounts = (counts + (PAD8 - 1)) // PAD8 * PAD8
    poff = jnp.cumsum(pcounts) - pcounts
    coff = jnp.cumsum(counts) - counts
    i_range = jnp.arange(TOKENS, dtype=jnp.int32)
    rank = i_range - jnp.take(coff, sorted_e)
    slot = jnp.take(poff, sorted_e) + rank
    gather_idx = jnp.zeros((TOKENS_PAD,), jnp.int32).at[slot].set(order)
    token_slot = jnp.zeros((TOKENS,), jnp.int32).at[order].set(slot)

    tiles_pe = (counts + (BLK - 1)) // BLK
    tcum = jnp.cumsum(tiles_pe)
    total = tcum[NUM_EXPERTS - 1]
    t_range = jnp.arange(MAX_TILES, dtype=jnp.int32)
    e_raw = jnp.searchsorted(tcum, t_range, side="right").astype(jnp.int32)
    e_cl = jnp.minimum(e_raw, NUM_EXPERTS - 1)
    i_t = t_range - (jnp.take(tcum, e_cl) - jnp.take(tiles_pe, e_cl))
    s_t = jnp.take(poff, e_cl) + i_t * BLK
    valid = jnp.clip(jnp.take(counts, e_cl) - i_t * BLK, 0, BLK)
    c_t = jnp.minimum(s_t, TOKENS_PAD - BLK)
    lo = s_t - c_t
    hi = lo + valid
    is_pad = t_range >= total
    e_last = sorted_e[TOKENS - 1]
    e_arr = jnp.where(is_pad, e_last, e_cl).astype(jnp.int32)
    c_arr = jnp.where(is_pad, 0, c_t).astype(jnp.int32)
    lo_arr = jnp.where(is_pad, 0, lo).astype(jnp.int32)
    hi_arr = jnp.where(is_pad, 0, hi).astype(jnp.int32)

    xs = _sc_gather_rows(x, gather_idx)
    os_ = _grouped_gemm(e_arr, c_arr, lo_arr, hi_arr, xs, w1, w3, w2)
    return _sc_gather_rows(os_, token_slot)
